# Initial kernel scaffold; baseline (speedup 1.0000x reference)
import jax
import jax.numpy as jnp
from jax.experimental import pallas as pl


def kernel(node_feats, edge_feats, edge_index, eW0, eb0, eW1, eb1, eW2, eb2, eW3, eb3, eW4, eb4, nW, nb, aW0, ab0, aW1, ab1):
    raise NotImplementedError("write your pallas kernel here")


# speedup vs baseline: 5.7338x; 5.7338x over previous
import functools

import jax
import jax.numpy as jnp
from jax import lax
from jax.experimental import pallas as pl
from jax.experimental.pallas import tpu as pltpu
from jax.experimental.pallas import tpu_sc as plsc

NC, NS = 2, 16
NW = NC * NS
SUMW = 4


def _silu(x):
    return x * jax.nn.sigmoid(x)


def _make_gather(N, E, D, CHUNK):
    EPW = E // NW
    NITER = EPW // CHUNK
    mesh = plsc.VectorSubcoreMesh(core_axis_name="c", subcore_axis_name="s",
                                  num_cores=NC, num_subcores=NS)

    @functools.partial(
        pl.kernel,
        out_type=(jax.ShapeDtypeStruct((E, D), jnp.float32),
                  jax.ShapeDtypeStruct((E, D), jnp.float32)),
        mesh=mesh,
        scratch_types=[
            pltpu.VMEM((CHUNK,), jnp.int32),
            pltpu.VMEM((CHUNK,), jnp.int32),
            pltpu.VMEM((CHUNK, D), jnp.float32),
            pltpu.VMEM((CHUNK, D), jnp.float32),
            pltpu.SemaphoreType.DMA,
            pltpu.SemaphoreType.DMA,
        ],
    )
    def gather_k(node_hbm, src_hbm, dst_hbm, src_out, dst_out,
                 idx_s, idx_d, rows_a, rows_b, sem_a, sem_b):
        cid = lax.axis_index("c")
        sid = lax.axis_index("s")
        wid = sid * NC + cid
        base = wid * EPW

        def body(i, carry):
            off = base + i * CHUNK
            pltpu.sync_copy(src_hbm.at[pl.ds(off, CHUNK)], idx_s)
            pltpu.sync_copy(dst_hbm.at[pl.ds(off, CHUNK)], idx_d)
            cp_a = pltpu.async_copy(node_hbm.at[idx_s], rows_a, sem_a)
            cp_a.wait()
            pltpu.sync_copy(rows_a, src_out.at[pl.ds(off, CHUNK)])
            cp_b = pltpu.async_copy(node_hbm.at[idx_d], rows_b, sem_b)
            cp_b.wait()
            pltpu.sync_copy(rows_b, dst_out.at[pl.ds(off, CHUNK)])
            return carry

        lax.fori_loop(0, NITER, body, 0)

    return gather_k


def _edge_mlp(ef, sf, df, w0e, w0s, w0d, b0, ws, bs, aw0, ab0, aw1r,
              E, D, TILE):
    G = E // TILE

    def body(ef_r, sf_r, df_r, w0e_r, w0s_r, w0d_r, b0_r,
             w1_r, b1_r, w2_r, b2_r, w3_r, b3_r, w4_r, b4_r,
             aw0_r, ab0_r, aw1r_r, h_out, s_out, m_out):
        x = jnp.dot(ef_r[...], w0e_r[...], preferred_element_type=jnp.float32)
        x = x + jnp.dot(sf_r[...], w0s_r[...], preferred_element_type=jnp.float32)
        x = x + jnp.dot(df_r[...], w0d_r[...], preferred_element_type=jnp.float32)
        h = _silu(x + b0_r[...])
        for w_r, b_r in ((w1_r, b1_r), (w2_r, b2_r), (w3_r, b3_r), (w4_r, b4_r)):
            h = _silu(jnp.dot(h, w_r[...], preferred_element_type=jnp.float32)
                      + b_r[...])
        h_out[...] = h
        a1 = _silu(jnp.dot(h, aw0_r[...], preferred_element_type=jnp.float32)
                   + ab0_r[...])
        s_row = lax.dot_general(aw1r_r[...], a1, (((1,), (1,)), ((), ())),
                                preferred_element_type=jnp.float32)
        s_out[...] = s_row.reshape(1, 1, s_row.shape[1])
        tmax = jnp.max(s_row)
        prev = jnp.where(pl.program_id(0) == 0, -jnp.inf, m_out[0, 0])
        m_out[0, 0] = jnp.maximum(prev, tmax)

    full = lambda i: (0, 0)
    wspec = pl.BlockSpec((D, D), full)
    bspec = pl.BlockSpec((1, D), full)
    tspec = pl.BlockSpec((TILE, D), lambda i: (i, 0))
    h, s1d, m = pl.pallas_call(
        body,
        grid=(G,),
        in_specs=[tspec, tspec, tspec,
                  wspec, wspec, wspec, bspec,
                  wspec, bspec, wspec, bspec, wspec, bspec, wspec, bspec,
                  wspec, bspec, bspec],
        out_specs=[tspec,
                   pl.BlockSpec((1, 1, TILE), lambda i: (i, 0, 0)),
                   pl.BlockSpec(memory_space=pltpu.SMEM)],
        out_shape=[jax.ShapeDtypeStruct((E, D), jnp.float32),
                   jax.ShapeDtypeStruct((G, 1, TILE), jnp.float32),
                   jax.ShapeDtypeStruct((1, 1), jnp.float32)],
    )(ef, sf, df, w0e, w0s, w0d, b0,
      ws[0], bs[0], ws[1], bs[1], ws[2], bs[2], ws[3], bs[3],
      aw0, ab0, aw1r)
    return h, s1d, m


def _weighting(h, m, aw0, ab0, aw1c, E, D, TILE):
    G = E // TILE

    def body(h_r, m_r, aw0_r, ab0_r, aw1c_r, w_out):
        hv = h_r[...]
        a1 = _silu(jnp.dot(hv, aw0_r[...], preferred_element_type=jnp.float32)
                   + ab0_r[...])
        s_col = jnp.dot(a1, aw1c_r[...], preferred_element_type=jnp.float32)
        p = jnp.exp(s_col - m_r[0, 0])
        w_out[...] = hv * p

    full = lambda i: (0, 0)
    tspec = pl.BlockSpec((TILE, D), lambda i: (i, 0))
    return pl.pallas_call(
        body,
        grid=(G,),
        in_specs=[tspec, pl.BlockSpec((1, 1), full),
                  pl.BlockSpec((D, D), full), pl.BlockSpec((1, D), full),
                  pl.BlockSpec((D, 1), full)],
        out_specs=tspec,
        out_shape=jax.ShapeDtypeStruct((E, D), jnp.float32),
    )(h, m, aw0, ab0, aw1c)


def _make_scatter(N, E, D, CHUNK, NPAD):
    EPW = E // NW
    NITER = EPW // CHUNK
    RPW = (N // NS) // 8 * 8
    TAIL = N - RPW * NS
    SFLAT = NPAD * SUMW
    SPW = SFLAT // NS
    assert CHUNK % 16 == 0 and (E // NW) % CHUNK == 0
    NVEC = CHUNK // 16
    mesh = plsc.VectorSubcoreMesh(core_axis_name="c", subcore_axis_name="s",
                                  num_cores=NC, num_subcores=NS)

    @functools.partial(
        pl.kernel,
        out_type=(jax.ShapeDtypeStruct((NC, N, D), jnp.float32),
                  jax.ShapeDtypeStruct((NC * SFLAT,), jnp.float32)),
        mesh=mesh,
        scratch_types=[
            pltpu.VMEM((CHUNK,), jnp.int32),
            pltpu.VMEM((CHUNK,), jnp.int32),
            pltpu.VMEM((CHUNK,), jnp.float32),
            pltpu.VMEM((CHUNK,), jnp.float32),
            pltpu.VMEM((CHUNK, D), jnp.float32),
            pltpu.VMEM((16,), jnp.float32),
            pltpu.VMEM_SHARED((N, D), jnp.float32),
            pltpu.VMEM_SHARED((SFLAT,), jnp.float32),
        ],
    )
    def scatter_k(w_hbm, s_hbm, dst_hbm, m_hbm, zrows_hbm, zflat_hbm,
                  agg_out, sum_out,
                  idx_v, idx8_v, s_v, p_v, rows_v, m_v, agg_sh, sum_sh):
        cid = lax.axis_index("c")
        sid = lax.axis_index("s")
        wid = sid * NC + cid

        pltpu.sync_copy(zrows_hbm, agg_sh.at[pl.ds(sid * RPW, RPW)])
        pltpu.sync_copy(zflat_hbm, sum_sh.at[pl.ds(sid * SPW, SPW)])

        @pl.when(sid == 0)
        def _():
            pltpu.sync_copy(zrows_hbm.at[pl.ds(0, TAIL)],
                            agg_sh.at[pl.ds(NS * RPW, TAIL)])
        pltpu.sync_copy(m_hbm, m_v)
        plsc.subcore_barrier()

        base = wid * EPW
        m_vec = m_v[...]

        def body(i, carry):
            off = base + i * CHUNK
            pltpu.sync_copy(dst_hbm.at[pl.ds(off, CHUNK)], idx_v)
            pltpu.sync_copy(s_hbm.at[pl.ds(off, CHUNK)], s_v)
            pltpu.sync_copy(w_hbm.at[pl.ds(off, CHUNK)], rows_v)

            def pbody(c, carry2):
                sl = pl.ds(c * 16, 16)
                p_v[sl] = jnp.exp(s_v[sl] - m_vec)
                idx8_v[sl] = idx_v[sl] * SUMW
                return carry2

            lax.fori_loop(0, NVEC, pbody, 0)
            pltpu.sync_copy(p_v, sum_sh.at[idx8_v], add=True)
            pltpu.sync_copy(rows_v, agg_sh.at[idx_v], add=True)
            return carry

        lax.fori_loop(0, NITER, body, 0)
        plsc.subcore_barrier()

        pltpu.sync_copy(agg_sh.at[pl.ds(sid * RPW, RPW)],
                        agg_out.at[cid, pl.ds(sid * RPW, RPW)])
        pltpu.sync_copy(sum_sh.at[pl.ds(sid * SPW, SPW)],
                        sum_out.at[pl.ds(cid * SFLAT + sid * SPW, SPW)])

        @pl.when(sid == 0)
        def _():
            pltpu.sync_copy(agg_sh.at[pl.ds(NS * RPW, TAIL)],
                            agg_out.at[cid, pl.ds(NS * RPW, TAIL)])

    return scatter_k


def _node_mlp(a0, a1, s0, s1, nf, nW, nb, N, D, TN):
    G = N // TN

    def body(a0_r, a1_r, s0_r, s1_r, nf_r, nw_r, nb_r, out_r):
        agg = a0_r[...] + a1_r[...]
        ssum = s0_r[:, 0:1] + s1_r[:, 0:1]
        x = agg / (ssum + 1e-20)
        out_r[...] = _silu(jnp.dot(x, nw_r[...],
                                   preferred_element_type=jnp.float32)
                           + nb_r[...]) + nf_r[...]

    full = lambda i: (0, 0)
    tspec = pl.BlockSpec((TN, D), lambda i: (i, 0))
    sspec = pl.BlockSpec((TN, SUMW), lambda i: (i, 0))
    return pl.pallas_call(
        body,
        grid=(G,),
        in_specs=[tspec, tspec, sspec, sspec, tspec,
                  pl.BlockSpec((D, D), full), pl.BlockSpec((1, D), full)],
        out_specs=tspec,
        out_shape=jax.ShapeDtypeStruct((N, D), jnp.float32),
    )(a0, a1, s0, s1, nf, nW, nb)


def kernel(node_feats, edge_feats, edge_index,
           eW0, eb0, eW1, eb1, eW2, eb2, eW3, eb3, eW4, eb4,
           nW, nb, aW0, ab0, aW1, ab1):
    N, D = node_feats.shape
    E = edge_feats.shape[0]
    TILE = 1280
    CHUNK = 400
    NPAD = 10240

    src = edge_index[0]
    dst = edge_index[1]

    src_rows, dst_rows = _make_gather(N, E, D, CHUNK)(node_feats, src, dst)

    w0e = eW0[:D]
    w0s = eW0[D:2 * D]
    w0d = eW0[2 * D:]
    b0 = eb0.reshape(1, D)
    ws = (eW1, eW2, eW3, eW4)
    bs = tuple(b.reshape(1, D) for b in (eb1, eb2, eb3, eb4))
    A = aW0.shape[1]
    aw0 = jnp.pad(aW0, ((0, 0), (0, D - A)))
    ab0p = jnp.pad(ab0, (0, D - A)).reshape(1, D)
    aw1c = jnp.pad(aW1, ((0, D - A), (0, 0)))
    aw1r = jnp.transpose(aw1c)

    h, s2, m = _edge_mlp(edge_feats, src_rows, dst_rows,
                         w0e, w0s, w0d, b0, ws, bs, aw0, ab0p, aw1r,
                         E, D, TILE)
    s1d = s2.reshape(E)

    weighted = _weighting(h, m, aw0, ab0p, aw1c, E, D, TILE)

    m16 = jnp.broadcast_to(m.reshape(1), (16,))
    zrows = jnp.zeros(((N // NS) // 8 * 8, D), jnp.float32)
    zflat = jnp.zeros((NPAD * SUMW // NS,), jnp.float32)
    aggp, sump = _make_scatter(N, E, D, 80, NPAD)(
        weighted, s1d, dst, m16, zrows, zflat)

    s2d = sump.reshape(NC, NPAD, SUMW)
    new_node = _node_mlp(aggp[0], aggp[1], s2d[0, :N], s2d[1, :N],
                         node_feats, nW, nb.reshape(1, D), N, D, 1000)
    return new_node, h
